# baseline (speedup 1.0000x reference)
import numpy as np
import jax
import jax.numpy as jnp
from jax.experimental import pallas as pl
from jax.experimental.pallas import tpu as pltpu

_BR = 1024
_NEG = float("-inf")
_STEPS = (1, 2, 4, 8, 16, 32, 64)


def _shift_consts():
    s_mats = np.zeros((len(_STEPS), 128, 128), dtype=np.float32)
    m_rows = np.zeros((len(_STEPS), 1, 128), dtype=np.float32)
    for k, s in enumerate(_STEPS):
        j = np.arange(128 - s)
        s_mats[k, j + s, j] = 1.0
        m_rows[k, 0, 128 - s:] = -np.inf
    return jnp.asarray(s_mats), jnp.asarray(m_rows)


def _revcummax_body(x_ref, s_ref, m_ref, o_ref):
    r = x_ref.shape[0]
    v = x_ref[...].reshape(r * 8, 128)
    for k in range(len(_STEPS)):
        shifted = jax.lax.dot(v, s_ref[k],
                              preferred_element_type=jnp.float32)
        v = jnp.maximum(v, shifted + m_ref[k])
    v = v.reshape(r, 8, 128)
    m = v[:, :, :1]
    for s in (1, 2, 4):
        m = jnp.maximum(
            m, jnp.pad(m[:, s:], ((0, 0), (0, s), (0, 0)),
                       constant_values=_NEG))
    e = jnp.pad(m[:, 1:], ((0, 0), (0, 1), (0, 0)), constant_values=_NEG)
    o_ref[...] = jnp.maximum(v, e)


@jax.jit
def kernel(x):
    b, c, h, w = x.shape
    flat = x.reshape(b * c * h, 8, w // 8)
    s_mats, m_rows = _shift_consts()
    n = len(_STEPS)
    out = pl.pallas_call(
        _revcummax_body,
        grid=(flat.shape[0] // _BR,),
        in_specs=[
            pl.BlockSpec((_BR, 8, w // 8), lambda i: (i, 0, 0)),
            pl.BlockSpec((n, 128, 128), lambda i: (0, 0, 0)),
            pl.BlockSpec((n, 1, 128), lambda i: (0, 0, 0)),
        ],
        out_specs=pl.BlockSpec((_BR, 8, w // 8), lambda i: (i, 0, 0)),
        out_shape=jax.ShapeDtypeStruct(flat.shape, flat.dtype),
        compiler_params=pltpu.CompilerParams(
            dimension_semantics=("arbitrary",)),
    )(flat, s_mats, m_rows)
    return out.reshape(b, c, h, w)


# speedup vs baseline: 1.7586x; 1.7586x over previous
import jax
import jax.numpy as jnp
from jax.experimental import pallas as pl
from jax.experimental.pallas import tpu as pltpu

_W = 1024
_BR = 1024
_HALVES = 2


def _revcummax_body(x_ref, o_ref):
    h = _BR // _HALVES
    parts = [x_ref[pl.ds(i * h, h), :] for i in range(_HALVES)]
    s = 1
    while s < _W:
        parts = [
            jnp.maximum(v, jnp.pad(v[:, s:], ((0, 0), (0, s)),
                                   constant_values=-jnp.inf))
            for v in parts
        ]
        s *= 2
    for i, v in enumerate(parts):
        o_ref[pl.ds(i * h, h), :] = v


@jax.jit
def kernel(x):
    b, c, h, w = x.shape
    flat = x.reshape(b * c * h, w)
    out = pl.pallas_call(
        _revcummax_body,
        grid=(flat.shape[0] // _BR,),
        in_specs=[pl.BlockSpec((_BR, w), lambda i: (i, 0))],
        out_specs=pl.BlockSpec((_BR, w), lambda i: (i, 0)),
        out_shape=jax.ShapeDtypeStruct(flat.shape, flat.dtype),
        compiler_params=pltpu.CompilerParams(
            dimension_semantics=("parallel",)),
    )(flat)
    return out.reshape(b, c, h, w)
